# baseline (speedup 1.0000x reference)
import functools

import jax
import jax.numpy as jnp
from jax.experimental import pallas as pl
from jax.experimental.pallas import tpu as pltpu

_THRESHOLD = 10.0
_NORM_FACTOR = 100.0
_PB = 256


def _dot_t(w, h):
    return jax.lax.dot_general(w, h, (((1,), (1,)), ((), ())),
                               preferred_element_type=jnp.float32)


def _fused_kernel(hl_ref, xl_ref, hp_ref, xp_ref, pm_ref,
                  aW1l_ref, aW1p_ref, aw1d_ref, ab1_ref,
                  aW2_ref, ab2_ref, aW3_ref, ab3_ref,
                  vW1p_ref, vw1d_ref, vb1_ref, vW2_ref, vb2_ref,
                  cW1l_ref, cW1p_ref, cw1d_ref, cb1_ref, cW2_ref, cb2_ref,
                  hout_ref, xout_ref):
    pj = pl.program_id(1)

    hl = hl_ref[0]
    xl = xl_ref[0]
    hp = hp_ref[0]
    xp = xp_ref[0]
    pm = pm_ref[0]

    L = hl.shape[0]
    P = hp.shape[0]

    Al = _dot_t(aW1l_ref[...], hl)
    Ap = _dot_t(aW1p_ref[...], hp)
    Vp = _dot_t(vW1p_ref[...], hp)
    Cl = _dot_t(cW1l_ref[...], hl)
    Cp = _dot_t(cW1p_ref[...], hp)

    rel0 = xl[:, 0:1] - xp[:, 0:1].reshape(1, P)
    rel1 = xl[:, 1:2] - xp[:, 1:2].reshape(1, P)
    rel2 = xl[:, 2:3] - xp[:, 2:3].reshape(1, P)
    d2 = rel0 * rel0 + rel1 * rel1 + rel2 * rel2

    edge = (jnp.sqrt(d2) < _THRESHOLD).astype(jnp.float32)
    dist = jnp.sqrt(d2 + 1e-8)
    inv = 1.0 / (dist + 1e-8)
    pm_row = pm.reshape(1, P)

    d2b = d2[None, :, :]

    a_h = jax.nn.silu(Al[:, :, None] + Ap[:, None, :]
                      + d2b * aw1d_ref[...][:, :, None]
                      + ab1_ref[...][:, :, None])
    a_h = jax.nn.silu(jnp.dot(aW2_ref[...], a_h.reshape(32, L * P),
                              preferred_element_type=jnp.float32)
                      + ab2_ref[...])
    a = jax.nn.sigmoid(jnp.dot(aW3_ref[...], a_h,
                               preferred_element_type=jnp.float32)
                       + ab3_ref[...])
    s = a.reshape(L, P) * pm_row * edge

    v_h = jax.nn.silu(Vp[:, None, :] + d2b * vw1d_ref[...][:, :, None]
                      + vb1_ref[...][:, :, None])
    v = (jnp.dot(vW2_ref[...], v_h.reshape(32, L * P),
                 preferred_element_type=jnp.float32)
         + vb2_ref[...]).reshape(32, L, P)
    h_contrib = jnp.sum(v * s[None, :, :], axis=2)

    c_h = jax.nn.silu(Cl[:, :, None] + Cp[:, None, :]
                      + d2b * cw1d_ref[...][:, :, None]
                      + cb1_ref[...][:, :, None])
    cw = jnp.tanh(jnp.dot(cW2_ref[...], c_h.reshape(32, L * P),
                          preferred_element_type=jnp.float32)
                  + cb2_ref[...])
    t = cw.reshape(L, P) * pm_row * edge * inv
    tsum = jnp.sum(t, axis=1, keepdims=True)
    x_contrib = xl * tsum - jnp.dot(t, xp, preferred_element_type=jnp.float32)

    @pl.when(pj == 0)
    def _init():
        hout_ref[0] = h_contrib
        xout_ref[0] = x_contrib

    @pl.when(pj != 0)
    def _acc():
        hout_ref[0] += h_contrib
        xout_ref[0] += x_contrib


@jax.jit
def kernel(h_ligand, x_ligand, h_protein, x_protein, ligand_mask, protein_mask,
           att_W1, att_b1, att_W2, att_b2, att_W3, att_b3,
           val_W1, val_b1, val_W2, val_b2,
           coord_W1, coord_b1, coord_W2, coord_b2):
    bs, n_lig, lig_nf = h_ligand.shape
    n_prot = h_protein.shape[1]
    hidden = att_W2.shape[0]

    aW1l = att_W1[:, :lig_nf]
    aW1p = att_W1[:, lig_nf:lig_nf + h_protein.shape[2]]
    aw1d = att_W1[:, -1:]
    vW1p = val_W1[:, :-1]
    vw1d = val_W1[:, -1:]
    cW1l = coord_W1[:, :lig_nf]
    cW1p = coord_W1[:, lig_nf:lig_nf + h_protein.shape[2]]
    cw1d = coord_W1[:, -1:]

    ab1 = att_b1.reshape(hidden, 1)
    ab2 = att_b2.reshape(hidden, 1)
    ab3 = att_b3.reshape(1, 1)
    vb1 = val_b1.reshape(hidden, 1)
    vb2 = val_b2.reshape(lig_nf, 1)
    cb1 = coord_b1.reshape(hidden, 1)
    cb2 = coord_b2.reshape(1, 1)

    n_pb = n_prot // _PB
    grid = (bs, n_pb)

    def full(a):
        return pl.BlockSpec(a.shape, lambda b, p: (0,) * a.ndim)

    hout, xout = pl.pallas_call(
        _fused_kernel,
        grid=grid,
        in_specs=[
            pl.BlockSpec((1, n_lig, lig_nf), lambda b, p: (b, 0, 0)),
            pl.BlockSpec((1, n_lig, 3), lambda b, p: (b, 0, 0)),
            pl.BlockSpec((1, _PB, h_protein.shape[2]), lambda b, p: (b, p, 0)),
            pl.BlockSpec((1, _PB, 3), lambda b, p: (b, p, 0)),
            pl.BlockSpec((1, _PB, 1), lambda b, p: (b, p, 0)),
            full(aW1l), full(aW1p), full(aw1d), full(ab1),
            full(att_W2), full(ab2), full(att_W3), full(ab3),
            full(vW1p), full(vw1d), full(vb1), full(val_W2), full(vb2),
            full(cW1l), full(cW1p), full(cw1d), full(cb1),
            full(coord_W2), full(cb2),
        ],
        out_specs=[
            pl.BlockSpec((1, lig_nf, n_lig), lambda b, p: (b, 0, 0)),
            pl.BlockSpec((1, n_lig, 3), lambda b, p: (b, 0, 0)),
        ],
        out_shape=[
            jax.ShapeDtypeStruct((bs, lig_nf, n_lig), jnp.float32),
            jax.ShapeDtypeStruct((bs, n_lig, 3), jnp.float32),
        ],
        compiler_params=pltpu.CompilerParams(
            dimension_semantics=("parallel", "arbitrary")),
    )(h_ligand, x_ligand, h_protein, x_protein, protein_mask,
      aW1l, aW1p, aw1d, ab1, att_W2, ab2, att_W3, ab3,
      vW1p, vw1d, vb1, val_W2, vb2,
      cW1l, cW1p, cw1d, cb1, coord_W2, cb2)

    h_cross = hout.transpose(0, 2, 1) * (ligand_mask / _NORM_FACTOR)
    x_cross = xout * (ligand_mask / _NORM_FACTOR)
    return (h_cross, x_cross)


# speedup vs baseline: 1.8623x; 1.8623x over previous
import jax
import jax.numpy as jnp
from jax.experimental import pallas as pl
from jax.experimental.pallas import tpu as pltpu

_THRESH2 = 100.0
_NORM_FACTOR = 100.0
_PB = 128
_L = 128


def _tile_lanes(x, n):
    while x.shape[-1] < n:
        x = jnp.concatenate([x, x], axis=-1)
    return x


def _sum_lane_tiles(x, n):
    while x.shape[-1] > n:
        h = x.shape[-1] // 2
        x = x[:, :h] + x[:, h:]
    return x


def _fused_kernel(hlT_ref, xlT_ref, hpT_ref, xpT_ref, pmT_ref,
                  W1_ref, W2_ref, aW3_ref, ab2_ref, ab3_ref, vb2_ref, cb2_ref,
                  hout_ref, xout_ref):
    pj = pl.program_id(1)

    hlT = hlT_ref[0]
    xlT = xlT_ref[0]
    hpT = hpT_ref[0]
    xpT = xpT_ref[0]
    pmT = pmT_ref[0]

    L = hlT.shape[1]
    P = hpT.shape[1]
    N = L * P

    sl = jnp.sum(xlT * xlT, axis=0, keepdims=True)
    sp = jnp.sum(xpT * xpT, axis=0, keepdims=True)
    ones_l = jnp.ones((1, L), jnp.float32)

    l_small = jnp.concatenate([hlT, xlT, sl, ones_l], axis=0)
    s_small = jnp.concatenate([hpT, xpT, sp, pmT], axis=0)

    l_t = _tile_lanes(l_small, N)
    s_s = jnp.repeat(s_small, L, axis=1)

    prod = l_t[32:35] * s_s[32:35]
    feat = jnp.concatenate([l_t, s_s, prod], axis=0)

    out97 = jnp.dot(W1_ref[...], feat, preferred_element_type=jnp.float32)

    d2 = out97[96:97]
    act = jax.nn.silu(out97[0:96])

    out65 = jnp.dot(W2_ref[...], act, preferred_element_type=jnp.float32)

    a_h = jax.nn.silu(out65[0:32] + ab2_ref[...])
    a = jax.nn.sigmoid(jnp.dot(aW3_ref[...], a_h,
                               preferred_element_type=jnp.float32)
                       + ab3_ref[...])
    v = out65[32:64] + vb2_ref[...]
    cw = jnp.tanh(out65[64:65] + cb2_ref[...])

    edge = (d2 < _THRESH2).astype(jnp.float32)
    dist = jnp.sqrt(d2 + 1e-8)
    inv = 1.0 / (dist + 1e-8)
    pe = s_s[36:37] * edge
    s = a * pe
    t = cw * pe * inv

    h_contrib = _sum_lane_tiles(v * s, L)

    rel = l_t[32:35] - s_s[32:35]
    x_contrib = _sum_lane_tiles(rel * t, L)

    @pl.when(pj == 0)
    def _init():
        hout_ref[0] = h_contrib
        xout_ref[0] = x_contrib

    @pl.when(pj != 0)
    def _acc():
        hout_ref[0] += h_contrib
        xout_ref[0] += x_contrib


@jax.jit
def kernel(h_ligand, x_ligand, h_protein, x_protein, ligand_mask, protein_mask,
           att_W1, att_b1, att_W2, att_b2, att_W3, att_b3,
           val_W1, val_b1, val_W2, val_b2,
           coord_W1, coord_b1, coord_W2, coord_b2):
    bs, n_lig, lig_nf = h_ligand.shape
    n_prot = h_protein.shape[1]
    prot_nf = h_protein.shape[2]
    hidden = att_W2.shape[0]
    f32 = jnp.float32

    def w1_rows(W1, b1):
        Wl = W1[:, :lig_nf] if W1.shape[1] == lig_nf + prot_nf + 1 else \
            jnp.zeros((hidden, lig_nf), f32)
        Wp = W1[:, -prot_nf - 1:-1]
        wd = W1[:, -1:]
        z3 = jnp.zeros((hidden, 3), f32)
        zc = jnp.zeros((hidden, 1), f32)
        return jnp.concatenate(
            [Wl, z3, wd, b1.reshape(hidden, 1),
             Wp, z3, wd, zc,
             jnp.broadcast_to(-2.0 * wd, (hidden, 3))], axis=1)

    d2_row = jnp.zeros((1, 77), f32).at[0, 35].set(1.0).at[0, 72].set(1.0) \
        .at[0, 74:77].set(-2.0)
    W1big = jnp.concatenate([
        w1_rows(att_W1, att_b1),
        w1_rows(val_W1, val_b1),
        w1_rows(coord_W1, coord_b1),
        d2_row], axis=0)

    z32 = jnp.zeros((hidden, hidden), f32)
    z1 = jnp.zeros((1, hidden), f32)
    W2big = jnp.concatenate([
        jnp.concatenate([att_W2, z32, z32], axis=1),
        jnp.concatenate([z32, val_W2, z32], axis=1),
        jnp.concatenate([z1, z1, coord_W2], axis=1)], axis=0)

    hlT = h_ligand.transpose(0, 2, 1)
    xlT = x_ligand.transpose(0, 2, 1)
    hpT = h_protein.transpose(0, 2, 1)
    xpT = x_protein.transpose(0, 2, 1)
    pmT = protein_mask.transpose(0, 2, 1)

    grid = (bs, n_prot // _PB)

    def full(shape):
        return pl.BlockSpec(shape, lambda b, p: (0,) * len(shape))

    hout, xout = pl.pallas_call(
        _fused_kernel,
        grid=grid,
        in_specs=[
            pl.BlockSpec((1, lig_nf, n_lig), lambda b, p: (b, 0, 0)),
            pl.BlockSpec((1, 3, n_lig), lambda b, p: (b, 0, 0)),
            pl.BlockSpec((1, prot_nf, _PB), lambda b, p: (b, 0, p)),
            pl.BlockSpec((1, 3, _PB), lambda b, p: (b, 0, p)),
            pl.BlockSpec((1, 1, _PB), lambda b, p: (b, 0, p)),
            full((97, 77)), full((65, 96)), full((1, hidden)),
            full((hidden, 1)), full((1, 1)), full((hidden, 1)), full((1, 1)),
        ],
        out_specs=[
            pl.BlockSpec((1, lig_nf, n_lig), lambda b, p: (b, 0, 0)),
            pl.BlockSpec((1, 3, n_lig), lambda b, p: (b, 0, 0)),
        ],
        out_shape=[
            jax.ShapeDtypeStruct((bs, lig_nf, n_lig), f32),
            jax.ShapeDtypeStruct((bs, 3, n_lig), f32),
        ],
        compiler_params=pltpu.CompilerParams(
            dimension_semantics=("parallel", "arbitrary")),
    )(hlT, xlT, hpT, xpT, pmT,
      W1big, W2big, att_W3,
      att_b2.reshape(hidden, 1), att_b3.reshape(1, 1),
      val_b2.reshape(lig_nf, 1), coord_b2.reshape(1, 1))

    h_cross = hout.transpose(0, 2, 1) * (ligand_mask / _NORM_FACTOR)
    x_cross = xout.transpose(0, 2, 1) * (ligand_mask / _NORM_FACTOR)
    return (h_cross, x_cross)
